# baseline (device time: 6710 ns/iter reference)
import jax
import jax.numpy as jnp
from jax import lax
from jax.experimental import pallas as pl
from jax.experimental.pallas import tpu as pltpu

X = 2


def kernel(x):
    m_per, n = x.shape
    n_loc = n // X

    def body(x_ref, out_ref, send_buf, send_sems, recv_sems):
        my_x = lax.axis_index("x")
        my_y = lax.axis_index("y")
        my_z = lax.axis_index("z")
        other = 1 - my_x
        half = m_per // 2

        send_buf[:, :] = x_ref[:, pl.ds(other * n_loc, n_loc)].astype(
            out_ref.dtype
        )

        barrier_sem = pltpu.get_barrier_semaphore()
        pl.semaphore_signal(
            barrier_sem, inc=1,
            device_id=(other, my_y, my_z),
            device_id_type=pl.DeviceIdType.MESH,
        )
        pl.semaphore_wait(barrier_sem, 1)

        rdmas = []
        for i in range(2):
            rdma = pltpu.make_async_remote_copy(
                src_ref=send_buf.at[pl.ds(i * half, half), :],
                dst_ref=out_ref.at[pl.ds(my_x * m_per + i * half, half), :],
                send_sem=send_sems.at[i],
                recv_sem=recv_sems.at[i],
                device_id=(other, my_y, my_z),
                device_id_type=pl.DeviceIdType.MESH,
            )
            rdma.start()
            rdmas.append(rdma)

        out_ref[pl.ds(my_x * m_per, m_per), :] = x_ref[
            :, pl.ds(my_x * n_loc, n_loc)
        ].astype(out_ref.dtype)

        for rdma in rdmas:
            rdma.wait()

    return pl.pallas_call(
        body,
        out_shape=jax.ShapeDtypeStruct((n, n_loc), jnp.bfloat16),
        in_specs=[pl.BlockSpec(memory_space=pltpu.VMEM)],
        out_specs=pl.BlockSpec(memory_space=pltpu.VMEM),
        scratch_shapes=[
            pltpu.VMEM((m_per, n_loc), jnp.bfloat16),
            pltpu.SemaphoreType.DMA((2,)),
            pltpu.SemaphoreType.DMA((2,)),
        ],
        compiler_params=pltpu.CompilerParams(collective_id=0),
    )(x)


# device time: 6672 ns/iter; 1.0057x vs baseline; 1.0057x over previous
import jax
import jax.numpy as jnp
from jax import lax
from jax.experimental import pallas as pl
from jax.experimental.pallas import tpu as pltpu

X = 2


def kernel(x):
    m_per, n = x.shape
    n_loc = n // X

    def body(x_ref, out_ref, send_buf, send_sem, recv_sem):
        my_x = lax.axis_index("x")
        my_y = lax.axis_index("y")
        my_z = lax.axis_index("z")
        other = 1 - my_x

        send_buf[:, :] = x_ref[:, pl.ds(other * n_loc, n_loc)].astype(
            out_ref.dtype
        )

        barrier_sem = pltpu.get_barrier_semaphore()
        pl.semaphore_signal(
            barrier_sem, inc=1,
            device_id=(other, my_y, my_z),
            device_id_type=pl.DeviceIdType.MESH,
        )
        pl.semaphore_wait(barrier_sem, 1)

        rdma = pltpu.make_async_remote_copy(
            src_ref=send_buf,
            dst_ref=out_ref.at[pl.ds(my_x * m_per, m_per), :],
            send_sem=send_sem,
            recv_sem=recv_sem,
            device_id=(other, my_y, my_z),
            device_id_type=pl.DeviceIdType.MESH,
        )
        rdma.start()

        out_ref[pl.ds(my_x * m_per, m_per), :] = x_ref[
            :, pl.ds(my_x * n_loc, n_loc)
        ].astype(out_ref.dtype)

        rdma.wait()

    return pl.pallas_call(
        body,
        out_shape=jax.ShapeDtypeStruct((n, n_loc), jnp.bfloat16),
        in_specs=[pl.BlockSpec(memory_space=pltpu.VMEM)],
        out_specs=pl.BlockSpec(memory_space=pltpu.VMEM),
        scratch_shapes=[
            pltpu.VMEM((m_per, n_loc), jnp.bfloat16),
            pltpu.SemaphoreType.DMA,
            pltpu.SemaphoreType.DMA,
        ],
        compiler_params=pltpu.CompilerParams(collective_id=0),
    )(x)
